# Initial kernel scaffold; baseline (speedup 1.0000x reference)
import jax
import jax.numpy as jnp
from jax.experimental import pallas as pl


def kernel(x, edge_index, loop_weight):
    raise NotImplementedError("write your pallas kernel here")


# speedup vs baseline: 4.1925x; 4.1925x over previous
import functools

import jax
import jax.numpy as jnp
from jax import lax
from jax.experimental import pallas as pl
from jax.experimental.pallas import tpu as pltpu
from jax.experimental.pallas import tpu_sc as plsc

N = 10000
E = 320000
D = 128

NC = 2
NS = 16
CH = 128
NW = NC * NS
CPT = -(-E // (CH * NW))
NCHUNK = CPT * NW
E_PAD = NCHUNK * CH
N_PAD = 10112
ZR = N_PAD // NS

_sc_mesh = plsc.VectorSubcoreMesh(core_axis_name="c", subcore_axis_name="s")


@functools.partial(
    pl.kernel,
    out_type=jax.ShapeDtypeStruct((NC, N_PAD, D), jnp.float32),
    mesh=_sc_mesh,
    scratch_types=[
        pltpu.VMEM((CH, D), jnp.float32),
        pltpu.VMEM((1, CH), jnp.int32),
        pltpu.VMEM((1, CH), jnp.int32),
        pltpu.VMEM_SHARED((N_PAD, D), jnp.float32),
        pltpu.SemaphoreType.DMA,
    ],
)
def _sc_scatter(x_hbm, srcc_hbm, dstc_hbm, zeros_hbm, out_hbm,
                rows_v, src_v, dst_v, agg_sh, sem):
    cid = lax.axis_index("c")
    sid = lax.axis_index("s")
    wid = cid * NS + sid

    pltpu.sync_copy(zeros_hbm, agg_sh.at[pl.ds(sid * ZR, ZR)])
    plsc.subcore_barrier()

    base = wid * CPT

    def body(j, carry):
        c = base + j
        pltpu.sync_copy(srcc_hbm.at[c], src_v.at[0])
        pltpu.sync_copy(dstc_hbm.at[c], dst_v.at[0])
        pltpu.async_copy(x_hbm.at[src_v.at[0]], rows_v, sem).wait()
        pltpu.sync_copy(rows_v, agg_sh.at[dst_v.at[0]], add=True)
        return carry

    lax.fori_loop(0, CPT, body, 0)
    plsc.subcore_barrier()

    pltpu.sync_copy(agg_sh.at[pl.ds(sid * ZR, ZR)],
                    out_hbm.at[cid, pl.ds(sid * ZR, ZR)])


def _tc_body(x_ref, w_ref, p_ref, o_ref):
    mm = jnp.dot(x_ref[...], w_ref[...], preferred_element_type=jnp.float32)
    o_ref[...] = jnp.maximum(p_ref[0] + p_ref[1] + mm, 0.0)


_BLK = 1000


def _tc_finish(x, w, partials):
    grid = (N // _BLK,)
    return pl.pallas_call(
        _tc_body,
        grid=grid,
        in_specs=[
            pl.BlockSpec((_BLK, D), lambda i: (i, 0)),
            pl.BlockSpec((D, D), lambda i: (0, 0)),
            pl.BlockSpec((NC, _BLK, D), lambda i: (0, i, 0)),
        ],
        out_specs=pl.BlockSpec((_BLK, D), lambda i: (i, 0)),
        out_shape=jax.ShapeDtypeStruct((N, D), jnp.float32),
    )(x, w, partials)


def kernel(x, edge_index, loop_weight):
    src = edge_index[0].astype(jnp.int32)
    dst = edge_index[1].astype(jnp.int32)
    pad = E_PAD - E
    src_c = jnp.concatenate([src, jnp.zeros((pad,), jnp.int32)]).reshape(NCHUNK, CH)
    dst_c = jnp.concatenate([dst, jnp.full((pad,), N, jnp.int32)]).reshape(NCHUNK, CH)
    zeros = jnp.zeros((ZR, D), jnp.float32)
    partials = _sc_scatter(x, src_c, dst_c, zeros)
    return _tc_finish(x, loop_weight, partials)


# speedup vs baseline: 5.1834x; 1.2363x over previous
import functools

import jax
import jax.numpy as jnp
from jax import lax
from jax.experimental import pallas as pl
from jax.experimental.pallas import tpu as pltpu
from jax.experimental.pallas import tpu_sc as plsc

N = 10000
E = 320000
D = 128
DH = D // 2

NC = 2
NS = 16
CH = 128
NBUF = 4
CPT = 160
E_PAD = CPT * NS * CH
N_PAD = 10112
ZR = N_PAD // NS

_sc_mesh = plsc.VectorSubcoreMesh(core_axis_name="c", subcore_axis_name="s")


@functools.partial(
    pl.kernel,
    out_type=jax.ShapeDtypeStruct((NC, N_PAD, DH), jnp.float32),
    mesh=_sc_mesh,
    compiler_params=pltpu.CompilerParams(use_tc_tiling_on_sc=False),
    scratch_types=[
        pltpu.VMEM((NBUF, CH, DH), jnp.float32),
        pltpu.VMEM((CPT, CH), jnp.int32),
        pltpu.VMEM((CPT, CH), jnp.int32),
        pltpu.VMEM_SHARED((N_PAD, DH), jnp.float32),
        pltpu.SemaphoreType.DMA,
        pltpu.SemaphoreType.DMA,
        pltpu.SemaphoreType.DMA,
        pltpu.SemaphoreType.DMA,
    ],
)
def _sc_scatter(x_hbm, srcc_hbm, dstc_hbm, zeros_hbm, out_hbm,
                rows_v, srci_v, dsti_v, agg_sh, *sems):
    cid = lax.axis_index("c")
    sid = lax.axis_index("s")

    pltpu.sync_copy(srcc_hbm.at[sid], srci_v)
    pltpu.sync_copy(dstc_hbm.at[sid], dsti_v)

    pltpu.sync_copy(zeros_hbm, agg_sh.at[pl.ds(sid * ZR, ZR)])
    plsc.subcore_barrier()

    xh = x_hbm.at[cid]

    for b in range(NBUF):
        pltpu.async_copy(xh.at[srci_v.at[b]], rows_v.at[b], sems[b])

    def group(g, carry):
        for b in range(NBUF):
            j = NBUF * g + b
            pltpu.make_async_copy(xh.at[srci_v.at[j]], rows_v.at[b], sems[b]).wait()
            pltpu.sync_copy(rows_v.at[b], agg_sh.at[dsti_v.at[j]], add=True)

            @pl.when(j + NBUF < CPT)
            def _():
                pltpu.async_copy(xh.at[srci_v.at[j + NBUF]], rows_v.at[b], sems[b])
        return carry

    lax.fori_loop(0, CPT // NBUF, group, 0)
    plsc.subcore_barrier()

    pltpu.sync_copy(agg_sh.at[pl.ds(sid * ZR, ZR)],
                    out_hbm.at[cid, pl.ds(sid * ZR, ZR)])


def _tc_body(x_ref, w_ref, p_ref, o_ref):
    mm = jnp.dot(x_ref[...], w_ref[...], preferred_element_type=jnp.float32)
    agg = jnp.concatenate([p_ref[0], p_ref[1]], axis=1)
    o_ref[...] = jnp.maximum(agg + mm, 0.0)


_BLK = 1000


def _tc_finish(x, w, partials):
    grid = (N // _BLK,)
    return pl.pallas_call(
        _tc_body,
        grid=grid,
        in_specs=[
            pl.BlockSpec((_BLK, D), lambda i: (i, 0)),
            pl.BlockSpec((D, D), lambda i: (0, 0)),
            pl.BlockSpec((NC, _BLK, DH), lambda i: (0, i, 0)),
        ],
        out_specs=pl.BlockSpec((_BLK, D), lambda i: (i, 0)),
        out_shape=jax.ShapeDtypeStruct((N, D), jnp.float32),
    )(x, w, partials)


def kernel(x, edge_index, loop_weight):
    src = edge_index[0].astype(jnp.int32)
    dst = edge_index[1].astype(jnp.int32)
    pad = E_PAD - E
    src_c = jnp.concatenate([src, jnp.zeros((pad,), jnp.int32)]).reshape(NS, CPT, CH)
    dst_c = jnp.concatenate([dst, jnp.full((pad,), N, jnp.int32)]).reshape(NS, CPT, CH)
    zeros = jnp.zeros((ZR, DH), jnp.float32)
    x_split = x.reshape(N, NC, DH).transpose(1, 0, 2)
    partials = _sc_scatter(x_split, src_c, dst_c, zeros)
    return _tc_finish(x, loop_weight, partials)
